# baseline (speedup 1.0000x reference)
import functools

import jax
import jax.numpy as jnp
from jax import lax
from jax.experimental import pallas as pl
from jax.experimental.pallas import tpu as pltpu
from jax.experimental.pallas import tpu_sc as plsc

VOCAB = 100000
DIM = 64
BATCH = 16384
SCALE = 10.0
EPS = 1e-8

_CHUNK = 16


def _sc_gather3(iword, oword, onword, center_table, context_table):
    info = plsc.get_sparse_core_info()
    nc, ns = info.num_cores, info.num_subcores
    nw = nc * ns
    b_per_w = BATCH // nw

    out_t = jax.ShapeDtypeStruct((BATCH, DIM), jnp.float32)

    @functools.partial(
        pl.kernel,
        out_type=[out_t, out_t, out_t],
        mesh=plsc.VectorSubcoreMesh(core_axis_name="c", subcore_axis_name="s"),
        scratch_types=[
            pltpu.VMEM((b_per_w,), jnp.int32),
            pltpu.SemaphoreType.DMA,
        ],
    )
    def gather_k(iw_hbm, ow_hbm, onw_hbm, ctr_hbm, ctx_hbm,
                 iv_hbm, ov_hbm, onv_hbm, idx_v, sem):
        wid = lax.axis_index("s") * nc + lax.axis_index("c")
        base = wid * b_per_w
        for idx_hbm, tab_hbm, out_hbm in (
            (iw_hbm, ctr_hbm, iv_hbm),
            (ow_hbm, ctx_hbm, ov_hbm),
            (onw_hbm, ctx_hbm, onv_hbm),
        ):
            pltpu.sync_copy(idx_hbm.at[pl.ds(base, b_per_w)], idx_v)

            def chunk(j, tab=tab_hbm, out=out_hbm):
                ivec = idx_v[pl.ds(j, _CHUNK)]
                copies = []
                for i in range(_CHUNK):
                    r = ivec[i]
                    copies.append(pltpu.async_copy(
                        tab.at[pl.ds(r, 1), :],
                        out.at[pl.ds(base + j + i, 1), :], sem))
                for c in copies:
                    c.wait()

            pl.loop(0, b_per_w, step=_CHUNK)(chunk)

    return gather_k(iword, oword, onword, center_table, context_table)


def _loss_body(iv_ref, ov_ref, onv_ref, out_ref):
    @pl.when(pl.program_id(0) == 0)
    def _init():
        out_ref[...] = jnp.zeros_like(out_ref)

    iv = iv_ref[...]
    ov = ov_ref[...]
    onv = onv_ref[...]
    ni = jnp.sqrt(jnp.sum(iv * iv, axis=1)) + EPS
    no = jnp.sqrt(jnp.sum(ov * ov, axis=1)) + EPS
    non = jnp.sqrt(jnp.sum(onv * onv, axis=1)) + EPS
    dio = jnp.sum(iv * ov, axis=1)
    dion = jnp.sum(iv * onv, axis=1)
    x1 = -SCALE * (1.0 - dio / (ni * no))
    x2 = SCALE * (1.0 - dion / (ni * non))
    oloss = jnp.minimum(x1, 0.0) - jnp.log1p(jnp.exp(-jnp.abs(x1)))
    nloss = jnp.minimum(x2, 0.0) - jnp.log1p(jnp.exp(-jnp.abs(x2)))
    total = -jnp.sum(oloss + nloss) / BATCH
    out_ref[...] += jnp.broadcast_to(total, (1, 1))


def kernel(iword, oword, onword, center_table, context_table):
    iword = iword.astype(jnp.int32)
    oword = oword.astype(jnp.int32)
    onword = onword.astype(jnp.int32)
    iv, ov, onv = _sc_gather3(iword, oword, onword, center_table, context_table)
    blk = 2048
    out = pl.pallas_call(
        _loss_body,
        grid=(BATCH // blk,),
        in_specs=[pl.BlockSpec((blk, DIM), lambda i: (i, 0))] * 3,
        out_specs=pl.BlockSpec((1, 1), lambda i: (0, 0)),
        out_shape=jax.ShapeDtypeStruct((1, 1), jnp.float32),
    )(iv, ov, onv)
    return out[0, 0]


# speedup vs baseline: 4.5338x; 4.5338x over previous
import functools

import jax
import jax.numpy as jnp
from jax import lax
from jax.experimental import pallas as pl
from jax.experimental.pallas import tpu as pltpu
from jax.experimental.pallas import tpu_sc as plsc

VOCAB = 100000
DIM = 64
BATCH = 16384
SCALE = 10.0
EPS = 1e-8

_CHUNK = 16


def _sc_gather3(iword, oword, onword, center_table, context_table):
    info = plsc.get_sparse_core_info()
    nc, ns = info.num_cores, info.num_subcores
    nw = nc * ns
    b_per_w = BATCH // nw

    out_t = jax.ShapeDtypeStruct((BATCH, DIM), jnp.float32)

    @functools.partial(
        pl.kernel,
        out_type=[out_t, out_t, out_t],
        mesh=plsc.VectorSubcoreMesh(core_axis_name="c", subcore_axis_name="s"),
        scratch_types=[
            pltpu.VMEM((b_per_w,), jnp.int32),
            pltpu.VMEM((b_per_w, DIM), jnp.float32),
            pltpu.SemaphoreType.DMA,
            pltpu.SemaphoreType.DMA,
        ],
    )
    def gather_k(iw_hbm, ow_hbm, onw_hbm, ctr_hbm, ctx_hbm,
                 iv_hbm, ov_hbm, onv_hbm, idx_v, rows_v, sem, osem):
        wid = lax.axis_index("s") * nc + lax.axis_index("c")
        base = wid * b_per_w
        for idx_hbm, tab_hbm, out_hbm in (
            (iw_hbm, ctr_hbm, iv_hbm),
            (ow_hbm, ctx_hbm, ov_hbm),
            (onw_hbm, ctx_hbm, onv_hbm),
        ):
            pltpu.sync_copy(idx_hbm.at[pl.ds(base, b_per_w)], idx_v)

            def chunk(j, tab=tab_hbm):
                ivec = idx_v[pl.ds(j, _CHUNK)]
                copies = []
                for i in range(_CHUNK):
                    r = ivec[i]
                    copies.append(pltpu.async_copy(
                        tab.at[pl.ds(r, 1), :],
                        rows_v.at[pl.ds(j + i, 1), :], sem))
                for c in copies:
                    c.wait()

            pl.loop(0, b_per_w, step=_CHUNK)(chunk)
            pltpu.async_copy(
                rows_v, out_hbm.at[pl.ds(base, b_per_w)], osem).wait()

    return gather_k(iword, oword, onword, center_table, context_table)


def _loss_body(iv_ref, ov_ref, onv_ref, out_ref):
    @pl.when(pl.program_id(0) == 0)
    def _init():
        out_ref[...] = jnp.zeros_like(out_ref)

    iv = iv_ref[...]
    ov = ov_ref[...]
    onv = onv_ref[...]
    ni = jnp.sqrt(jnp.sum(iv * iv, axis=1)) + EPS
    no = jnp.sqrt(jnp.sum(ov * ov, axis=1)) + EPS
    non = jnp.sqrt(jnp.sum(onv * onv, axis=1)) + EPS
    dio = jnp.sum(iv * ov, axis=1)
    dion = jnp.sum(iv * onv, axis=1)
    x1 = -SCALE * (1.0 - dio / (ni * no))
    x2 = SCALE * (1.0 - dion / (ni * non))
    oloss = jnp.minimum(x1, 0.0) - jnp.log1p(jnp.exp(-jnp.abs(x1)))
    nloss = jnp.minimum(x2, 0.0) - jnp.log1p(jnp.exp(-jnp.abs(x2)))
    total = -jnp.sum(oloss + nloss) / BATCH
    out_ref[...] += jnp.broadcast_to(total, (1, 1))


def kernel(iword, oword, onword, center_table, context_table):
    iword = iword.astype(jnp.int32)
    oword = oword.astype(jnp.int32)
    onword = onword.astype(jnp.int32)
    iv, ov, onv = _sc_gather3(iword, oword, onword, center_table, context_table)
    blk = 2048
    out = pl.pallas_call(
        _loss_body,
        grid=(BATCH // blk,),
        in_specs=[pl.BlockSpec((blk, DIM), lambda i: (i, 0))] * 3,
        out_specs=pl.BlockSpec((1, 1), lambda i: (0, 0)),
        out_shape=jax.ShapeDtypeStruct((1, 1), jnp.float32),
    )(iv, ov, onv)
    return out[0, 0]


# speedup vs baseline: 5.3687x; 1.1842x over previous
import functools

import jax
import jax.numpy as jnp
from jax import lax
from jax.experimental import pallas as pl
from jax.experimental.pallas import tpu as pltpu
from jax.experimental.pallas import tpu_sc as plsc

VOCAB = 100000
DIM = 64
BATCH = 16384
SCALE = 10.0
EPS = 1e-8

_CHUNK = 16


def _sc_gather3(iword, oword, onword, center_table, context_table):
    info = plsc.get_sparse_core_info()
    nc, ns = info.num_cores, info.num_subcores
    nw = nc * ns
    b_per_w = BATCH // nw

    out_t = jax.ShapeDtypeStruct((BATCH, DIM), jnp.float32)

    @functools.partial(
        pl.kernel,
        out_type=[out_t, out_t, out_t],
        mesh=plsc.VectorSubcoreMesh(core_axis_name="c", subcore_axis_name="s"),
        scratch_types=[
            pltpu.VMEM((b_per_w,), jnp.int32),
            pltpu.VMEM((b_per_w, DIM), jnp.float32),
            pltpu.SemaphoreType.DMA,
            pltpu.SemaphoreType.DMA,
        ],
    )
    def gather_k(iw_hbm, ow_hbm, onw_hbm, ctr_hbm, ctx_hbm,
                 iv_hbm, ov_hbm, onv_hbm, idx_v, rows, sem, osem):
        wid = lax.axis_index("s") * nc + lax.axis_index("c")
        base = wid * b_per_w
        for idx_hbm, tab_hbm, out_hbm in (
            (iw_hbm, ctr_hbm, iv_hbm),
            (ow_hbm, ctx_hbm, ov_hbm),
            (onw_hbm, ctx_hbm, onv_hbm),
        ):
            pltpu.sync_copy(idx_hbm.at[pl.ds(base, b_per_w)], idx_v)

            def issue(j, tab=tab_hbm):
                ivec = idx_v[pl.ds(j, _CHUNK)]
                for i in range(_CHUNK):
                    pltpu.async_copy(
                        tab.at[pl.ds(ivec[i], 1), :],
                        rows.at[pl.ds(j + i, 1), :], sem)

            def drain(j, tab=tab_hbm):
                for i in range(_CHUNK):
                    pltpu.make_async_copy(
                        tab.at[pl.ds(0, 1), :],
                        rows.at[pl.ds(j + i, 1), :], sem).wait()

            def body(j, issue=issue, drain=drain):
                issue(j)
                drain(j - _CHUNK)

            issue(0)
            pl.loop(_CHUNK, b_per_w, step=_CHUNK)(body)
            drain(b_per_w - _CHUNK)
            pltpu.async_copy(
                rows, out_hbm.at[pl.ds(base, b_per_w)], osem).wait()

    return gather_k(iword, oword, onword, center_table, context_table)


def _loss_body(iv_ref, ov_ref, onv_ref, out_ref):
    @pl.when(pl.program_id(0) == 0)
    def _init():
        out_ref[...] = jnp.zeros_like(out_ref)

    iv = iv_ref[...]
    ov = ov_ref[...]
    onv = onv_ref[...]
    ni = jnp.sqrt(jnp.sum(iv * iv, axis=1)) + EPS
    no = jnp.sqrt(jnp.sum(ov * ov, axis=1)) + EPS
    non = jnp.sqrt(jnp.sum(onv * onv, axis=1)) + EPS
    dio = jnp.sum(iv * ov, axis=1)
    dion = jnp.sum(iv * onv, axis=1)
    x1 = -SCALE * (1.0 - dio / (ni * no))
    x2 = SCALE * (1.0 - dion / (ni * non))
    oloss = jnp.minimum(x1, 0.0) - jnp.log1p(jnp.exp(-jnp.abs(x1)))
    nloss = jnp.minimum(x2, 0.0) - jnp.log1p(jnp.exp(-jnp.abs(x2)))
    total = -jnp.sum(oloss + nloss) / BATCH
    out_ref[...] += jnp.broadcast_to(total, (1, 1))


def kernel(iword, oword, onword, center_table, context_table):
    iword = iword.astype(jnp.int32)
    oword = oword.astype(jnp.int32)
    onword = onword.astype(jnp.int32)
    iv, ov, onv = _sc_gather3(iword, oword, onword, center_table, context_table)
    blk = 2048
    out = pl.pallas_call(
        _loss_body,
        grid=(BATCH // blk,),
        in_specs=[pl.BlockSpec((blk, DIM), lambda i: (i, 0))] * 3,
        out_specs=pl.BlockSpec((1, 1), lambda i: (0, 0)),
        out_shape=jax.ShapeDtypeStruct((1, 1), jnp.float32),
    )(iv, ov, onv)
    return out[0, 0]


# speedup vs baseline: 5.4060x; 1.0069x over previous
import functools

import jax
import jax.numpy as jnp
from jax import lax
from jax.experimental import pallas as pl
from jax.experimental.pallas import tpu as pltpu
from jax.experimental.pallas import tpu_sc as plsc

VOCAB = 100000
DIM = 64
BATCH = 16384
SCALE = 10.0
EPS = 1e-8

_CHUNK = 16


def _sc_gather3(iword, oword, onword, center_table, context_table):
    info = plsc.get_sparse_core_info()
    nc, ns = info.num_cores, info.num_subcores
    nw = nc * ns
    b_per_w = BATCH // nw

    out_t = jax.ShapeDtypeStruct((BATCH, DIM), jnp.float32)

    @functools.partial(
        pl.kernel,
        out_type=[out_t, out_t, out_t],
        mesh=plsc.VectorSubcoreMesh(core_axis_name="c", subcore_axis_name="s"),
        scratch_types=[
            pltpu.VMEM((b_per_w,), jnp.int32),
            pltpu.VMEM((b_per_w, DIM), jnp.float32),
            pltpu.SemaphoreType.DMA,
            pltpu.SemaphoreType.DMA,
        ],
    )
    def gather_k(iw_hbm, ow_hbm, onw_hbm, ctr_hbm, ctx_hbm,
                 iv_hbm, ov_hbm, onv_hbm, idx_v, rows, sem, osem):
        wid = lax.axis_index("s") * nc + lax.axis_index("c")
        base = wid * b_per_w
        for idx_hbm, tab_hbm, out_hbm in (
            (iw_hbm, ctr_hbm, iv_hbm),
            (ow_hbm, ctx_hbm, ov_hbm),
            (onw_hbm, ctx_hbm, onv_hbm),
        ):
            pltpu.sync_copy(idx_hbm.at[pl.ds(base, b_per_w)], idx_v)

            def issue(j, tab=tab_hbm):
                ivec = idx_v[pl.ds(j, _CHUNK)]
                for i in range(_CHUNK):
                    pltpu.async_copy(
                        tab.at[pl.ds(ivec[i], 1), :],
                        rows.at[pl.ds(j + i, 1), :], sem)

            def drain(j, tab=tab_hbm):
                pltpu.make_async_copy(
                    tab.at[pl.ds(0, _CHUNK), :],
                    rows.at[pl.ds(j, _CHUNK), :], sem).wait()

            def body(j, issue=issue, drain=drain):
                issue(j)
                drain(j - _CHUNK)

            issue(0)
            pl.loop(_CHUNK, b_per_w, step=_CHUNK)(body)
            drain(b_per_w - _CHUNK)
            pltpu.async_copy(
                rows, out_hbm.at[pl.ds(base, b_per_w)], osem).wait()

    return gather_k(iword, oword, onword, center_table, context_table)


def _loss_body(iv_ref, ov_ref, onv_ref, out_ref):
    @pl.when(pl.program_id(0) == 0)
    def _init():
        out_ref[...] = jnp.zeros_like(out_ref)

    iv = iv_ref[...]
    ov = ov_ref[...]
    onv = onv_ref[...]
    ni = jnp.sqrt(jnp.sum(iv * iv, axis=1)) + EPS
    no = jnp.sqrt(jnp.sum(ov * ov, axis=1)) + EPS
    non = jnp.sqrt(jnp.sum(onv * onv, axis=1)) + EPS
    dio = jnp.sum(iv * ov, axis=1)
    dion = jnp.sum(iv * onv, axis=1)
    x1 = -SCALE * (1.0 - dio / (ni * no))
    x2 = SCALE * (1.0 - dion / (ni * non))
    oloss = jnp.minimum(x1, 0.0) - jnp.log1p(jnp.exp(-jnp.abs(x1)))
    nloss = jnp.minimum(x2, 0.0) - jnp.log1p(jnp.exp(-jnp.abs(x2)))
    total = -jnp.sum(oloss + nloss) / BATCH
    out_ref[...] += jnp.broadcast_to(total, (1, 1))


def kernel(iword, oword, onword, center_table, context_table):
    iword = iword.astype(jnp.int32)
    oword = oword.astype(jnp.int32)
    onword = onword.astype(jnp.int32)
    iv, ov, onv = _sc_gather3(iword, oword, onword, center_table, context_table)
    blk = 2048
    out = pl.pallas_call(
        _loss_body,
        grid=(BATCH // blk,),
        in_specs=[pl.BlockSpec((blk, DIM), lambda i: (i, 0))] * 3,
        out_specs=pl.BlockSpec((1, 1), lambda i: (0, 0)),
        out_shape=jax.ShapeDtypeStruct((1, 1), jnp.float32),
    )(iv, ov, onv)
    return out[0, 0]


# speedup vs baseline: 5.4658x; 1.0111x over previous
import functools

import jax
import jax.numpy as jnp
from jax import lax
from jax.experimental import pallas as pl
from jax.experimental.pallas import tpu as pltpu
from jax.experimental.pallas import tpu_sc as plsc

VOCAB = 100000
DIM = 64
BATCH = 16384
SCALE = 10.0
EPS = 1e-8

_VCH = 2048
_NG = -(-VOCAB // _VCH)
_PROWS = _NG * _VCH // 2


def _transpose_pack_body(ctrT_ref, ctxT_ref, ctr_ref, ctx_ref):
    h = _VCH // 2
    for src, dst in ((ctrT_ref, ctr_ref), (ctxT_ref, ctx_ref)):
        x = src[...]
        dst[...] = jnp.concatenate(
            [jnp.transpose(x[:, :h]), jnp.transpose(x[:, h:])], axis=1)


def _transpose_pack(ctrT, ctxT):
    out_t = jax.ShapeDtypeStruct((_PROWS, 2 * DIM), jnp.float32)
    return pl.pallas_call(
        _transpose_pack_body,
        grid=(_NG,),
        in_specs=[pl.BlockSpec((DIM, _VCH), lambda i: (0, i))] * 2,
        out_specs=[pl.BlockSpec((_VCH // 2, 2 * DIM), lambda i: (i, 0))] * 2,
        out_shape=[out_t, out_t],
    )(ctrT, ctxT)


def _sc_gather3(iword, oword, onword, ctr_pairs, ctx_pairs):
    info = plsc.get_sparse_core_info()
    nc, ns = info.num_cores, info.num_subcores
    nw = nc * ns
    b_per_w = BATCH // nw
    half = b_per_w // 2

    out_t = jax.ShapeDtypeStruct((BATCH, 2 * DIM), jnp.float32)

    @functools.partial(
        pl.kernel,
        out_type=[out_t, out_t, out_t],
        mesh=plsc.VectorSubcoreMesh(core_axis_name="c", subcore_axis_name="s"),
        scratch_types=[
            pltpu.VMEM((b_per_w,), jnp.int32),
            pltpu.VMEM((b_per_w,), jnp.int32),
            pltpu.VMEM((half, 2 * DIM), jnp.float32),
            pltpu.SemaphoreType.DMA,
            pltpu.SemaphoreType.DMA,
        ],
    )
    def gather_k(iw_hbm, ow_hbm, onw_hbm, ctr_hbm, ctx_hbm,
                 iv_hbm, ov_hbm, onv_hbm, idx_v, pidx_v, pairs_v, sem, osem):
        wid = lax.axis_index("s") * nc + lax.axis_index("c")
        base = wid * b_per_w
        for idx_hbm, tab_hbm, out_hbm in (
            (iw_hbm, ctr_hbm, iv_hbm),
            (ow_hbm, ctx_hbm, ov_hbm),
            (onw_hbm, ctx_hbm, onv_hbm),
        ):
            pltpu.sync_copy(idx_hbm.at[pl.ds(base, b_per_w)], idx_v)

            def to_pairs(j):
                r = idx_v[pl.ds(j, 16)]
                pidx_v[pl.ds(j, 16)] = jax.lax.shift_left(
                    jax.lax.shift_right_logical(r, 11), 10) + jnp.bitwise_and(
                        r, 1023)

            pl.loop(0, b_per_w, step=16)(to_pairs)
            for h in range(2):
                pltpu.async_copy(
                    tab_hbm.at[pidx_v.at[pl.ds(h * half, half)]],
                    pairs_v, sem).wait()
                pltpu.async_copy(
                    pairs_v,
                    out_hbm.at[pl.ds(base + h * half, half)], osem).wait()

    return gather_k(iword, oword, onword, ctr_pairs, ctx_pairs)


def _loss_body(iv_ref, ov_ref, onv_ref, pi_ref, po_ref, pon_ref, out_ref):
    @pl.when(pl.program_id(0) == 0)
    def _init():
        out_ref[...] = jnp.zeros_like(out_ref)

    def sel(x_ref, p_ref):
        x = x_ref[...]
        p = p_ref[...]
        return x[:, :DIM] * (1.0 - p) + x[:, DIM:] * p

    iv = sel(iv_ref, pi_ref)
    ov = sel(ov_ref, po_ref)
    onv = sel(onv_ref, pon_ref)
    ni = jnp.sqrt(jnp.sum(iv * iv, axis=1)) + EPS
    no = jnp.sqrt(jnp.sum(ov * ov, axis=1)) + EPS
    non = jnp.sqrt(jnp.sum(onv * onv, axis=1)) + EPS
    dio = jnp.sum(iv * ov, axis=1)
    dion = jnp.sum(iv * onv, axis=1)
    x1 = -SCALE * (1.0 - dio / (ni * no))
    x2 = SCALE * (1.0 - dion / (ni * non))
    oloss = jnp.minimum(x1, 0.0) - jnp.log1p(jnp.exp(-jnp.abs(x1)))
    nloss = jnp.minimum(x2, 0.0) - jnp.log1p(jnp.exp(-jnp.abs(x2)))
    total = -jnp.sum(oloss + nloss) / BATCH
    out_ref[...] += jnp.broadcast_to(total, (1, 1))


def kernel(iword, oword, onword, center_table, context_table):
    iword = iword.astype(jnp.int32)
    oword = oword.astype(jnp.int32)
    onword = onword.astype(jnp.int32)
    ctr_pairs, ctx_pairs = _transpose_pack(center_table.T, context_table.T)
    iv, ov, onv = _sc_gather3(iword, oword, onword, ctr_pairs, ctx_pairs)
    par = [
        jnp.bitwise_and(jax.lax.shift_right_logical(w, 10), 1)
        .astype(jnp.float32).reshape(BATCH, 1)
        for w in (iword, oword, onword)
    ]
    blk = 2048
    out = pl.pallas_call(
        _loss_body,
        grid=(BATCH // blk,),
        in_specs=[pl.BlockSpec((blk, 2 * DIM), lambda i: (i, 0))] * 3
        + [pl.BlockSpec((blk, 1), lambda i: (i, 0))] * 3,
        out_specs=pl.BlockSpec((1, 1), lambda i: (0, 0)),
        out_shape=jax.ShapeDtypeStruct((1, 1), jnp.float32),
    )(iv, ov, onv, *par)
    return out[0, 0]


# speedup vs baseline: 5.9135x; 1.0819x over previous
import functools

import jax
import jax.numpy as jnp
from jax import lax
from jax.experimental import pallas as pl
from jax.experimental.pallas import tpu as pltpu
from jax.experimental.pallas import tpu_sc as plsc

VOCAB = 100000
DIM = 64
BATCH = 16384
SCALE = 10.0
EPS = 1e-8

_VCH = 4096
_NG = -(-VOCAB // _VCH)
_PROWS = _NG * _VCH // 2


def _transpose_pack_body(ctrT_ref, ctxT_ref, ctr_ref, ctx_ref):
    h = _VCH // 2
    for src, dst in ((ctrT_ref, ctr_ref), (ctxT_ref, ctx_ref)):
        x = src[...]
        dst[...] = jnp.concatenate(
            [jnp.transpose(x[:, :h]), jnp.transpose(x[:, h:])], axis=1)


def _transpose_pack(ctrT, ctxT):
    out_t = jax.ShapeDtypeStruct((_PROWS, 2 * DIM), jnp.float32)
    return pl.pallas_call(
        _transpose_pack_body,
        grid=(_NG,),
        in_specs=[pl.BlockSpec((DIM, _VCH), lambda i: (0, i))] * 2,
        out_specs=[pl.BlockSpec((_VCH // 2, 2 * DIM), lambda i: (i, 0))] * 2,
        out_shape=[out_t, out_t],
    )(ctrT, ctxT)


def _sc_gather3(iword, oword, onword, ctr_pairs, ctx_pairs):
    info = plsc.get_sparse_core_info()
    nc, ns = info.num_cores, info.num_subcores
    nw = nc * ns
    b_per_w = BATCH // nw
    half = b_per_w // 2

    out_t = jax.ShapeDtypeStruct((BATCH, 2 * DIM), jnp.float32)

    @functools.partial(
        pl.kernel,
        out_type=[out_t, out_t, out_t],
        mesh=plsc.VectorSubcoreMesh(core_axis_name="c", subcore_axis_name="s"),
        scratch_types=[
            pltpu.VMEM((b_per_w,), jnp.int32),
            pltpu.VMEM((b_per_w,), jnp.int32),
            pltpu.VMEM((half, 2 * DIM), jnp.float32),
            pltpu.SemaphoreType.DMA,
            pltpu.SemaphoreType.DMA,
        ],
    )
    def gather_k(iw_hbm, ow_hbm, onw_hbm, ctr_hbm, ctx_hbm,
                 iv_hbm, ov_hbm, onv_hbm, idx_v, pidx_v, pairs_v, sem, osem):
        wid = lax.axis_index("s") * nc + lax.axis_index("c")
        base = wid * b_per_w
        for idx_hbm, tab_hbm, out_hbm in (
            (iw_hbm, ctr_hbm, iv_hbm),
            (ow_hbm, ctx_hbm, ov_hbm),
            (onw_hbm, ctx_hbm, onv_hbm),
        ):
            pltpu.sync_copy(idx_hbm.at[pl.ds(base, b_per_w)], idx_v)

            def to_pairs(j):
                r = idx_v[pl.ds(j, 16)]
                pidx_v[pl.ds(j, 16)] = jax.lax.shift_left(
                    jax.lax.shift_right_logical(r, 12), 11) + jnp.bitwise_and(
                        r, 2047)

            pl.loop(0, b_per_w, step=16)(to_pairs)
            for h in range(2):
                pltpu.async_copy(
                    tab_hbm.at[pidx_v.at[pl.ds(h * half, half)]],
                    pairs_v, sem).wait()
                pltpu.async_copy(
                    pairs_v,
                    out_hbm.at[pl.ds(base + h * half, half)], osem).wait()

    return gather_k(iword, oword, onword, ctr_pairs, ctx_pairs)


def _loss_body(iv_ref, ov_ref, onv_ref, pi_ref, po_ref, pon_ref, out_ref):
    @pl.when(pl.program_id(0) == 0)
    def _init():
        out_ref[...] = jnp.zeros_like(out_ref)

    def sel(x_ref, p_ref):
        x = x_ref[...]
        p = p_ref[...]
        return x[:, :DIM] * (1.0 - p) + x[:, DIM:] * p

    iv = sel(iv_ref, pi_ref)
    ov = sel(ov_ref, po_ref)
    onv = sel(onv_ref, pon_ref)
    ni = jnp.sqrt(jnp.sum(iv * iv, axis=1)) + EPS
    no = jnp.sqrt(jnp.sum(ov * ov, axis=1)) + EPS
    non = jnp.sqrt(jnp.sum(onv * onv, axis=1)) + EPS
    dio = jnp.sum(iv * ov, axis=1)
    dion = jnp.sum(iv * onv, axis=1)
    x1 = -SCALE * (1.0 - dio / (ni * no))
    x2 = SCALE * (1.0 - dion / (ni * non))
    oloss = jnp.minimum(x1, 0.0) - jnp.log1p(jnp.exp(-jnp.abs(x1)))
    nloss = jnp.minimum(x2, 0.0) - jnp.log1p(jnp.exp(-jnp.abs(x2)))
    total = -jnp.sum(oloss + nloss) / BATCH
    out_ref[...] += jnp.broadcast_to(total, (1, 1))


def kernel(iword, oword, onword, center_table, context_table):
    iword = iword.astype(jnp.int32)
    oword = oword.astype(jnp.int32)
    onword = onword.astype(jnp.int32)
    ctr_pairs, ctx_pairs = _transpose_pack(center_table.T, context_table.T)
    iv, ov, onv = _sc_gather3(iword, oword, onword, ctr_pairs, ctx_pairs)
    par = [
        jnp.bitwise_and(jax.lax.shift_right_logical(w, 11), 1)
        .astype(jnp.float32).reshape(BATCH, 1)
        for w in (iword, oword, onword)
    ]
    blk = 4096
    out = pl.pallas_call(
        _loss_body,
        grid=(BATCH // blk,),
        in_specs=[pl.BlockSpec((blk, 2 * DIM), lambda i: (i, 0))] * 3
        + [pl.BlockSpec((blk, 1), lambda i: (i, 0))] * 3,
        out_specs=pl.BlockSpec((1, 1), lambda i: (0, 0)),
        out_shape=jax.ShapeDtypeStruct((1, 1), jnp.float32),
    )(iv, ov, onv, *par)
    return out[0, 0]


# speedup vs baseline: 6.1259x; 1.0359x over previous
import functools

import jax
import jax.numpy as jnp
from jax import lax
from jax.experimental import pallas as pl
from jax.experimental.pallas import tpu as pltpu
from jax.experimental.pallas import tpu_sc as plsc

VOCAB = 100000
DIM = 64
BATCH = 16384
SCALE = 10.0
EPS = 1e-8

_VCH = 8192
_NG = -(-VOCAB // _VCH)
_PROWS = _NG * _VCH // 2


def _transpose_pack_body(ctrT_ref, ctxT_ref, ctr_ref, ctx_ref):
    h = _VCH // 2
    for src, dst in ((ctrT_ref, ctr_ref), (ctxT_ref, ctx_ref)):
        x = src[...]
        dst[...] = jnp.concatenate(
            [jnp.transpose(x[:, :h]), jnp.transpose(x[:, h:])], axis=1)


def _transpose_pack(ctrT, ctxT):
    out_t = jax.ShapeDtypeStruct((_PROWS, 2 * DIM), jnp.float32)
    return pl.pallas_call(
        _transpose_pack_body,
        grid=(_NG,),
        in_specs=[pl.BlockSpec((DIM, _VCH), lambda i: (0, i))] * 2,
        out_specs=[pl.BlockSpec((_VCH // 2, 2 * DIM), lambda i: (i, 0))] * 2,
        out_shape=[out_t, out_t],
    )(ctrT, ctxT)


def _sc_gather3(iword, oword, onword, ctr_pairs, ctx_pairs):
    info = plsc.get_sparse_core_info()
    nc, ns = info.num_cores, info.num_subcores
    nw = nc * ns
    b_per_w = BATCH // nw
    half = b_per_w // 2

    out_t = jax.ShapeDtypeStruct((BATCH, 2 * DIM), jnp.float32)

    @functools.partial(
        pl.kernel,
        out_type=[out_t, out_t, out_t],
        mesh=plsc.VectorSubcoreMesh(core_axis_name="c", subcore_axis_name="s"),
        scratch_types=[
            pltpu.VMEM((b_per_w,), jnp.int32),
            pltpu.VMEM((b_per_w,), jnp.int32),
            pltpu.VMEM((half, 2 * DIM), jnp.float32),
            pltpu.SemaphoreType.DMA,
            pltpu.SemaphoreType.DMA,
        ],
    )
    def gather_k(iw_hbm, ow_hbm, onw_hbm, ctr_hbm, ctx_hbm,
                 iv_hbm, ov_hbm, onv_hbm, idx_v, pidx_v, pairs_v, sem, osem):
        wid = lax.axis_index("s") * nc + lax.axis_index("c")
        base = wid * b_per_w
        for idx_hbm, tab_hbm, out_hbm in (
            (iw_hbm, ctr_hbm, iv_hbm),
            (ow_hbm, ctx_hbm, ov_hbm),
            (onw_hbm, ctx_hbm, onv_hbm),
        ):
            pltpu.sync_copy(idx_hbm.at[pl.ds(base, b_per_w)], idx_v)

            def to_pairs(j):
                r = idx_v[pl.ds(j, 16)]
                pidx_v[pl.ds(j, 16)] = jax.lax.shift_left(
                    jax.lax.shift_right_logical(r, 13), 12) + jnp.bitwise_and(
                        r, 4095)

            pl.loop(0, b_per_w, step=16)(to_pairs)
            for h in range(2):
                pltpu.async_copy(
                    tab_hbm.at[pidx_v.at[pl.ds(h * half, half)]],
                    pairs_v, sem).wait()
                pltpu.async_copy(
                    pairs_v,
                    out_hbm.at[pl.ds(base + h * half, half)], osem).wait()

    return gather_k(iword, oword, onword, ctr_pairs, ctx_pairs)


def _loss_body(iv_ref, ov_ref, onv_ref, pi_ref, po_ref, pon_ref, out_ref):
    @pl.when(pl.program_id(0) == 0)
    def _init():
        out_ref[...] = jnp.zeros_like(out_ref)

    def sel(x_ref, p_ref):
        x = x_ref[...]
        p = p_ref[...]
        return x[:, :DIM] * (1.0 - p) + x[:, DIM:] * p

    iv = sel(iv_ref, pi_ref)
    ov = sel(ov_ref, po_ref)
    onv = sel(onv_ref, pon_ref)
    ni = jnp.sqrt(jnp.sum(iv * iv, axis=1)) + EPS
    no = jnp.sqrt(jnp.sum(ov * ov, axis=1)) + EPS
    non = jnp.sqrt(jnp.sum(onv * onv, axis=1)) + EPS
    dio = jnp.sum(iv * ov, axis=1)
    dion = jnp.sum(iv * onv, axis=1)
    x1 = -SCALE * (1.0 - dio / (ni * no))
    x2 = SCALE * (1.0 - dion / (ni * non))
    oloss = jnp.minimum(x1, 0.0) - jnp.log1p(jnp.exp(-jnp.abs(x1)))
    nloss = jnp.minimum(x2, 0.0) - jnp.log1p(jnp.exp(-jnp.abs(x2)))
    total = -jnp.sum(oloss + nloss) / BATCH
    out_ref[...] += jnp.broadcast_to(total, (1, 1))


def kernel(iword, oword, onword, center_table, context_table):
    iword = iword.astype(jnp.int32)
    oword = oword.astype(jnp.int32)
    onword = onword.astype(jnp.int32)
    ctr_pairs, ctx_pairs = _transpose_pack(center_table.T, context_table.T)
    iv, ov, onv = _sc_gather3(iword, oword, onword, ctr_pairs, ctx_pairs)
    par = [
        jnp.bitwise_and(jax.lax.shift_right_logical(w, 12), 1)
        .astype(jnp.float32).reshape(BATCH, 1)
        for w in (iword, oword, onword)
    ]
    blk = 4096
    out = pl.pallas_call(
        _loss_body,
        grid=(BATCH // blk,),
        in_specs=[pl.BlockSpec((blk, 2 * DIM), lambda i: (i, 0))] * 3
        + [pl.BlockSpec((blk, 1), lambda i: (i, 0))] * 3,
        out_specs=pl.BlockSpec((1, 1), lambda i: (0, 0)),
        out_shape=jax.ShapeDtypeStruct((1, 1), jnp.float32),
    )(iv, ov, onv, *par)
    return out[0, 0]
